# baseline (speedup 1.0000x reference)
import functools

import jax
import jax.numpy as jnp
import numpy as np
from jax import lax
from jax.experimental import pallas as pl
from jax.experimental.pallas import tpu as pltpu
from jax.experimental.pallas import tpu_sc as plsc

_F32 = jnp.float32


def _table_body(embl_ref, embr_ref, w_ref, b_ref, out_ref):
    embl = embl_ref[...]
    embr = embr_ref[...]
    w = w_ref[...]
    b = b_ref[...]

    u12 = lax.broadcasted_iota(jnp.int32, (60, 12), 0)
    k12 = lax.broadcasted_iota(jnp.int32, (60, 12), 1)
    rowsel = (u12 % 12 == k12).astype(_F32)
    tmp = lax.dot_general(rowsel, w, (((1,), (1,)), ((), ())),
                          preferred_element_type=_F32, precision=lax.Precision.HIGHEST)
    c3 = lax.broadcasted_iota(jnp.int32, (3, 16), 0)
    v3 = lax.broadcasted_iota(jnp.int32, (3, 16), 1)
    colsel = (v3 % 3 == c3).astype(_F32)
    wtile = jnp.dot(tmp, colsel, preferred_element_type=_F32, precision=lax.Precision.HIGHEST)
    u16 = lax.broadcasted_iota(jnp.int32, (60, 16), 0)
    v16 = lax.broadcasted_iota(jnp.int32, (60, 16), 1)
    bigw = jnp.where((u16 // 12) == (v16 // 3), wtile, 0.0)

    el = jnp.dot(embl, bigw, preferred_element_type=_F32, precision=lax.Precision.HIGHEST)
    er = jnp.dot(embr, bigw, preferred_element_type=_F32, precision=lax.Precision.HIGHEST)

    i25 = lax.broadcasted_iota(jnp.int32, (25, 5), 0)
    j25 = lax.broadcasted_iota(jnp.int32, (25, 5), 1)
    oa = (i25 // 5 == j25).astype(_F32)
    ob = (i25 % 5 == j25).astype(_F32)

    bt = jnp.dot(b, colsel, preferred_element_type=_F32, precision=lax.Precision.HIGHEST)
    vmask = (lax.broadcasted_iota(jnp.int32, (1, 16), 1) < 15).astype(_F32)

    out_ref[...] = (jnp.dot(oa, el, preferred_element_type=_F32, precision=lax.Precision.HIGHEST)
                    + jnp.dot(ob, er, preferred_element_type=_F32, precision=lax.Precision.HIGHEST)
                    + bt * vmask)


def _build_table(embl, embr, w, b2):
    return pl.pallas_call(
        _table_body,
        out_shape=jax.ShapeDtypeStruct((25, 16), _F32),
    )(embl, embr, w, b2)


_N_PAIRS = 8192
_N_ROWS = 5 * _N_PAIRS


def _sc_gather(x, tflat):
    info = plsc.get_sparse_core_info()
    nc, ns = info.num_cores, info.num_subcores
    nw = nc * ns
    pairs_per_w = _N_PAIRS // nw
    nblk = pairs_per_w // 16
    xchunk = pairs_per_w * 2
    rows_per_w = pairs_per_w * 5

    mesh = plsc.VectorSubcoreMesh(core_axis_name="c", subcore_axis_name="s")

    @functools.partial(
        pl.kernel,
        out_type=jax.ShapeDtypeStruct((3 * _N_ROWS,), _F32),
        mesh=mesh,
        compiler_params=pltpu.CompilerParams(needs_layout_passes=False),
        scratch_types=[
            pltpu.VMEM((xchunk,), jnp.int32),
            pltpu.VMEM((400,), _F32),
            pltpu.VMEM((3 * rows_per_w,), _F32),
            pltpu.SemaphoreType.DMA,
        ],
    )
    def body(x_hbm, t_hbm, out_hbm, x_v, t_v, out_v, sem):
        wid = lax.axis_index("s") * nc + lax.axis_index("c")
        lane = lax.iota(jnp.int32, 16)
        cx = pltpu.make_async_copy(x_hbm.at[pl.ds(wid * xchunk, xchunk)], x_v, sem)
        cx.start()
        ct = pltpu.make_async_copy(t_hbm, t_v, sem)
        ct.start()
        cx.wait()
        ct.wait()
        for t in range(nblk):
            xe = plsc.load_gather(x_v, [lane * 2 + 32 * t])
            xo = plsc.load_gather(x_v, [lane * 2 + (32 * t + 1)])
            pv16 = xe * 80 + xo * 16
            for r in range(15):
                vals = plsc.load_gather(t_v, [pv16 + r])
                out_v[pl.ds(r * pairs_per_w + 16 * t, 16)] = vals
        copies = []
        for r in range(15):
            copies.append(pltpu.make_async_copy(
                out_v.at[pl.ds(r * pairs_per_w, pairs_per_w)],
                out_hbm.at[pl.ds(r * _N_PAIRS + wid * pairs_per_w, pairs_per_w)],
                sem,
            ))
            copies[-1].start()
        for cp in copies:
            cp.wait()

    return body(x, tflat)


def kernel(x, emb, W, b):
    x = x.astype(jnp.int32)
    emb2 = emb.reshape(5, 30).astype(_F32)
    embl = jnp.pad(emb2, ((0, 0), (0, 30)))
    embr = jnp.pad(emb2, ((0, 0), (30, 0)))
    b2 = b.reshape(1, 3).astype(_F32)
    table = _build_table(embl, embr, W.astype(_F32), b2)
    out_t = _sc_gather(x, table.reshape(400))
    return out_t.reshape(15, _N_PAIRS).T.reshape(_N_ROWS, 3)


# speedup vs baseline: 1.8057x; 1.8057x over previous
import functools

import jax
import jax.numpy as jnp
import numpy as np
from jax import lax
from jax.experimental import pallas as pl
from jax.experimental.pallas import tpu as pltpu
from jax.experimental.pallas import tpu_sc as plsc

_F32 = jnp.float32


def _table_body(embl_ref, embr_ref, w_ref, b_ref, out_ref):
    embl = embl_ref[...]
    embr = embr_ref[...]
    w = w_ref[...]
    b = b_ref[...]

    u12 = lax.broadcasted_iota(jnp.int32, (60, 12), 0)
    k12 = lax.broadcasted_iota(jnp.int32, (60, 12), 1)
    rowsel = (u12 % 12 == k12).astype(_F32)
    tmp = lax.dot_general(rowsel, w, (((1,), (1,)), ((), ())),
                          preferred_element_type=_F32, precision=lax.Precision.HIGHEST)
    c3 = lax.broadcasted_iota(jnp.int32, (3, 16), 0)
    v3 = lax.broadcasted_iota(jnp.int32, (3, 16), 1)
    colsel = (v3 % 3 == c3).astype(_F32)
    wtile = jnp.dot(tmp, colsel, preferred_element_type=_F32, precision=lax.Precision.HIGHEST)
    u16 = lax.broadcasted_iota(jnp.int32, (60, 16), 0)
    v16 = lax.broadcasted_iota(jnp.int32, (60, 16), 1)
    bigw = jnp.where((u16 // 12) == (v16 // 3), wtile, 0.0)

    el = jnp.dot(embl, bigw, preferred_element_type=_F32, precision=lax.Precision.HIGHEST)
    er = jnp.dot(embr, bigw, preferred_element_type=_F32, precision=lax.Precision.HIGHEST)

    i25 = lax.broadcasted_iota(jnp.int32, (25, 5), 0)
    j25 = lax.broadcasted_iota(jnp.int32, (25, 5), 1)
    oa = (i25 // 5 == j25).astype(_F32)
    ob = (i25 % 5 == j25).astype(_F32)

    bt = jnp.dot(b, colsel, preferred_element_type=_F32, precision=lax.Precision.HIGHEST)
    vmask = (lax.broadcasted_iota(jnp.int32, (1, 16), 1) < 15).astype(_F32)

    out_ref[...] = (jnp.dot(oa, el, preferred_element_type=_F32, precision=lax.Precision.HIGHEST)
                    + jnp.dot(ob, er, preferred_element_type=_F32, precision=lax.Precision.HIGHEST)
                    + bt * vmask)


def _build_table(embl, embr, w, b2):
    return pl.pallas_call(
        _table_body,
        out_shape=jax.ShapeDtypeStruct((25, 16), _F32),
    )(embl, embr, w, b2)


_N_PAIRS = 8192
_N_ROWS = 5 * _N_PAIRS


def _sc_gather(x, tflat, consts):
    info = plsc.get_sparse_core_info()
    nc, ns = info.num_cores, info.num_subcores
    nw = nc * ns
    pairs_per_w = _N_PAIRS // nw
    nblk = pairs_per_w // 16
    xchunk = pairs_per_w * 2
    rows_per_w = pairs_per_w * 5

    mesh = plsc.VectorSubcoreMesh(core_axis_name="c", subcore_axis_name="s")

    @functools.partial(
        pl.kernel,
        out_type=jax.ShapeDtypeStruct((3 * _N_ROWS,), _F32),
        mesh=mesh,
        compiler_params=pltpu.CompilerParams(needs_layout_passes=False),
        scratch_types=[
            pltpu.VMEM((xchunk,), jnp.int32),
            pltpu.VMEM((400,), _F32),
            pltpu.VMEM((160,), jnp.int32),
            pltpu.VMEM((pairs_per_w,), jnp.int32),
            pltpu.VMEM((3 * rows_per_w,), _F32),
            pltpu.SemaphoreType.DMA,
        ],
    )
    def body(x_hbm, t_hbm, k_hbm, out_hbm, x_v, t_v, k_v, p_v, out_v, sem):
        wid = lax.axis_index("s") * nc + lax.axis_index("c")
        lane = lax.iota(jnp.int32, 16)
        cx = pltpu.make_async_copy(x_hbm.at[pl.ds(wid * xchunk, xchunk)], x_v, sem)
        cx.start()
        ct = pltpu.make_async_copy(t_hbm, t_v, sem)
        ct.start()
        ck = pltpu.make_async_copy(k_hbm, k_v, sem)
        ck.start()
        cx.wait()
        ct.wait()
        ck.wait()
        for t in range(nblk):
            xe = plsc.load_gather(x_v, [lane * 2 + 32 * t])
            xo = plsc.load_gather(x_v, [lane * 2 + (32 * t + 1)])
            p_v[pl.ds(16 * t, 16)] = xe * 80 + xo * 16
        jpat = [k_v[pl.ds(16 * s, 16)] for s in range(5)]
        vpatc = [[k_v[pl.ds(80 + 16 * s, 16)] + c for c in range(3)]
                 for s in range(5)]
        for q in range(nblk):
            for s in range(5):
                pv = plsc.load_gather(p_v, [jpat[s] + 16 * q])
                for c in range(3):
                    vals = plsc.load_gather(t_v, [pv + vpatc[s][c]])
                    out_v[pl.ds(c * rows_per_w + 80 * q + 16 * s, 16)] = vals
        copies = []
        for c in range(3):
            copies.append(pltpu.make_async_copy(
                out_v.at[pl.ds(c * rows_per_w, rows_per_w)],
                out_hbm.at[pl.ds(c * _N_ROWS + wid * rows_per_w, rows_per_w)],
                sem,
            ))
            copies[-1].start()
        for cp in copies:
            cp.wait()

    return body(x, tflat, consts)


def kernel(x, emb, W, b):
    x = x.astype(jnp.int32)
    emb2 = emb.reshape(5, 30).astype(_F32)
    embl = jnp.pad(emb2, ((0, 0), (0, 30)))
    embr = jnp.pad(emb2, ((0, 0), (30, 0)))
    b2 = b.reshape(1, 3).astype(_F32)
    table = _build_table(embl, embr, W.astype(_F32), b2)
    rows80 = np.arange(80)
    consts = jnp.asarray(
        np.concatenate([rows80 // 5, 3 * (rows80 % 5)]).astype(np.int32))
    out_t = _sc_gather(x, table.reshape(400), consts)
    return out_t.reshape(3, _N_ROWS).T
